# baseline (speedup 1.0000x reference)
import functools

import jax
import jax.numpy as jnp
from jax import lax
from jax.experimental import pallas as pl
from jax.experimental.pallas import tpu as pltpu
from jax.experimental.pallas import tpu_sc as plsc

NC = 2
NS = 16
L = 16
NW = NC * NS

B = 262144
NLEV = 6
T_RES = (2, 2, 4, 4, 8, 8)
S_RES = (16, 24, 32, 48, 64, 80)
SIZES = tuple(T_RES[i] * S_RES[i] ** 3 for i in range(NLEV))

CHUNK = 128
PTS_PER_W = B // NW
NCHUNK = PTS_PER_W // CHUNK
GRP = CHUNK // L

RESIDENT = (True, True, False, False, False, False)

_mesh = plsc.VectorSubcoreMesh(core_axis_name="c", subcore_axis_name="s",
                               num_cores=NC, num_subcores=NS)
_cparams = pltpu.CompilerParams(
    needs_layout_passes=False, use_tc_tiling_on_sc=False)


_DET_R = 8192

def _det_plan(size):
    rpt = size // NW
    for r in (8192, 8000, 6912, 4096, 3456, 2304, 2048, 1024, 864, 512, 256):
        if r <= rpt and rpt % r == 0:
            return rpt, r, rpt // r
    return rpt, rpt, 1

_DET = tuple(_det_plan(s) for s in SIZES)


def _detile_body(w0, w1, w2, w3, w4, w5, o0, o1, o2, o3, o4, o5, buf, fb):
    ws = (w0, w1, w2, w3, w4, w5)
    os = (o0, o1, o2, o3, o4, o5)
    wid = lax.axis_index("s") * NC + lax.axis_index("c")
    iota = lax.broadcasted_iota(jnp.int32, (L,), 0)
    zv = jnp.full((L,), 0, dtype=jnp.int32)
    ov = jnp.full((L,), 1, dtype=jnp.int32)

    for t in range(NLEV):
        rpt, r, k = _DET[t]
        base = wid * rpt
        for c in range(k):
            r0 = base + c * r
            pltpu.sync_copy(ws[t].at[pl.ds(r0, r), :], buf.at[pl.ds(0, r), :])

            def grp(g, cy, r=r):
                rows = g * L + iota
                e0 = plsc.load_gather(buf, [rows, zv])
                e1 = plsc.load_gather(buf, [rows, ov])
                plsc.store_scatter(fb, [rows * 2], e0)
                plsc.store_scatter(fb, [rows * 2 + 1], e1)
                return cy

            lax.fori_loop(0, r // L, grp, 0)
            pltpu.sync_copy(fb.at[pl.ds(0, 2 * r)],
                            os[t].at[pl.ds(2 * r0, 2 * r)])


_detile = functools.partial(
    pl.kernel,
    out_type=tuple(jax.ShapeDtypeStruct((2 * s,), jnp.float32)
                   for s in SIZES),
    mesh=_mesh,
    compiler_params=_cparams,
    scratch_types=[
        pltpu.VMEM((_DET_R, 2), jnp.float32),
        pltpu.VMEM((2 * _DET_R,), jnp.float32),
    ],
)(_detile_body)


def _cell(xb4, g, lev, iota):
    rs = (T_RES[lev] - 1, S_RES[lev] - 1, S_RES[lev] - 1, S_RES[lev] - 1)
    rows = g * L + iota
    bli = []
    wts = []
    for c in range(4):
        xv = plsc.load_gather(xb4, [rows, jnp.full((L,), c, jnp.int32)])
        f = xv * jnp.float32(rs[c])
        b = f.astype(jnp.int32)
        b = jnp.minimum(b, rs[c] - 1)
        bli.append(b)
        wts.append(f - b.astype(jnp.float32))
    return bli, wts


def _vertex_indices(bli, lev, scale):
    X = S_RES[lev]
    SY = X
    SZ = X * X
    ST = X * X * X
    base = (bli[0] * (ST * scale) + bli[1] * scale
            + bli[2] * (SY * scale) + bli[3] * (SZ * scale))
    idxs = []
    for i in (0, 1):
        for j in (0, 1):
            for k in (0, 1):
                for l in (0, 1):
                    off = (i * ST + j + k * SY + l * SZ) * scale
                    idxs.append(base + off)
    return idxs


def _interp(e, wts):
    wt, wx, wy, wz = wts
    a = [e[m] + wt * (e[8 + m] - e[m]) for m in range(8)]
    b = [a[m] + wx * (a[4 + m] - a[m]) for m in range(4)]
    c = [b[m] + wy * (b[2 + m] - b[m]) for m in range(2)]
    return c[0] + wz * (c[1] - c[0])


def _body(x, w0f, w1f, w2f, w3f, w4f, w5f, out, mask, tab0, tab1, xb4, idxb,
          rowb, outb, maskb, sem):
    ws = (None, None, w2f, w3f, w4f, w5f)
    wid = lax.axis_index("s") * NC + lax.axis_index("c")
    base = wid * PTS_PER_W
    iota = lax.broadcasted_iota(jnp.int32, (L,), 0)

    pltpu.sync_copy(w0f, tab0)
    pltpu.sync_copy(w1f, tab1)
    tabs = (tab0, tab1)

    def fullv(v):
        return jnp.full((L,), v, dtype=jnp.int32)

    def process_chunk(ci, carry):
        off = base + ci * CHUNK
        pltpu.sync_copy(x.at[pl.ds(off, CHUNK), :], xb4)

        def mask_grp(g, c2):
            rows = g * L + iota
            ok = None
            for c in range(4):
                xv = plsc.load_gather(xb4, [rows, fullv(c)])
                okc = (xv >= jnp.float32(0.0)) & (xv <= jnp.float32(1.0))
                ok = okc if ok is None else (ok & okc)
            maskb[pl.ds(g * L, L)] = jnp.where(ok, jnp.int32(1), jnp.int32(0))
            return c2

        lax.fori_loop(0, GRP, mask_grp, 0)

        for lev in range(NLEV):
            if RESIDENT[lev]:
                tab = tabs[lev]

                def grp_res(g, c2, lev=lev, tab=tab):
                    bli, wts = _cell(xb4, g, lev, iota)
                    vidx2 = _vertex_indices(bli, lev, 2)
                    rows = g * L + iota
                    for f in (0, 1):
                        e = [plsc.load_gather(tab, [vidx2[v] + f])
                             for v in range(16)]
                        o = _interp(e, wts)
                        plsc.store_scatter(outb, [rows, fullv(2 * lev + f)],
                                           o)
                    return c2

                lax.fori_loop(0, GRP, grp_res, 0)
            else:
                wl = ws[lev]

                def grp_idx(g, c2, lev=lev):
                    bli, _ = _cell(xb4, g, lev, iota)
                    vidx2 = _vertex_indices(bli, lev, 2)
                    for v in range(16):
                        idxb[pl.ds((2 * v) * CHUNK + g * L, L)] = vidx2[v]
                        idxb[pl.ds((2 * v + 1) * CHUNK + g * L, L)] = (
                            vidx2[v] + 1)
                    return c2

                lax.fori_loop(0, GRP, grp_idx, 0)
                pltpu.async_copy(wl.at[idxb], rowb, sem).wait()

                def grp_int(g, c2, lev=lev):
                    _, wts = _cell(xb4, g, lev, iota)
                    rows = g * L + iota
                    for f in (0, 1):
                        e = [rowb[pl.ds((2 * v + f) * CHUNK + g * L, L)]
                             for v in range(16)]
                        o = _interp(e, wts)
                        plsc.store_scatter(outb, [rows, fullv(2 * lev + f)],
                                           o)
                    return c2

                lax.fori_loop(0, GRP, grp_int, 0)

        pltpu.sync_copy(outb, out.at[pl.ds(off, CHUNK), :])
        pltpu.sync_copy(maskb, mask.at[pl.ds(off, CHUNK)])
        return carry

    lax.fori_loop(0, NCHUNK, process_chunk, 0)


_hash_embed = functools.partial(
    pl.kernel,
    out_type=(
        jax.ShapeDtypeStruct((B, 12), jnp.float32),
        jax.ShapeDtypeStruct((B,), jnp.int32),
    ),
    mesh=_mesh,
    compiler_params=_cparams,
    scratch_types=[
        pltpu.VMEM((SIZES[0] * 2,), jnp.float32),
        pltpu.VMEM((SIZES[1] * 2,), jnp.float32),
        pltpu.VMEM((CHUNK, 4), jnp.float32),
        pltpu.VMEM((32 * CHUNK,), jnp.int32),
        pltpu.VMEM((32 * CHUNK,), jnp.float32),
        pltpu.VMEM((CHUNK, 12), jnp.float32),
        pltpu.VMEM((CHUNK,), jnp.int32),
        pltpu.SemaphoreType.DMA,
    ],
)(_body)


def kernel(x, w0, w1, w2, w3, w4, w5):
    wf = _detile(w0, w1, w2, w3, w4, w5)
    out, mask = _hash_embed(x, *wf)
    return out, mask.astype(bool)


# speedup vs baseline: 1.4995x; 1.4995x over previous
import functools

import jax
import jax.numpy as jnp
from jax import lax
from jax.experimental import pallas as pl
from jax.experimental.pallas import tpu as pltpu
from jax.experimental.pallas import tpu_sc as plsc

NC = 2
NS = 16
L = 16
NW = NC * NS

B = 262144
NLEV = 6
T_RES = (2, 2, 4, 4, 8, 8)
S_RES = (16, 24, 32, 48, 64, 80)
SIZES = tuple(T_RES[i] * S_RES[i] ** 3 for i in range(NLEV))

CHUNK = 128
PTS_PER_W = B // NW
NCHUNK = PTS_PER_W // CHUNK
GRP = CHUNK // L

RESIDENT = (True, True, False, False, False, False)

_mesh = plsc.VectorSubcoreMesh(core_axis_name="c", subcore_axis_name="s",
                               num_cores=NC, num_subcores=NS)
_cparams = pltpu.CompilerParams(
    needs_layout_passes=False, use_tc_tiling_on_sc=False)


def _tc_split_body(w_ref, a_ref, b_ref):
    blk = w_ref[...]
    a_ref[...] = blk[:, 0]
    b_ref[...] = blk[:, 1]


def _tc_split(w):
    size = w.shape[0]
    rb = size
    for cand in (4096, 3072, 2048, 1024):
        if size % cand == 0:
            rb = cand
            break
    return pl.pallas_call(
        _tc_split_body,
        grid=(size // rb,),
        in_specs=[pl.BlockSpec((rb, 2), lambda i: (i, 0))],
        out_specs=(pl.BlockSpec((rb,), lambda i: (i,)),
                   pl.BlockSpec((rb,), lambda i: (i,))),
        out_shape=(jax.ShapeDtypeStruct((size,), jnp.float32),
                   jax.ShapeDtypeStruct((size,), jnp.float32)),
    )(w)


def _tc_stack_body(*refs):
    cols = [r[...] for r in refs[:12]]
    refs[12][...] = jnp.stack(cols, axis=-1)


def _tc_stack(cols):
    rb = 1024
    return pl.pallas_call(
        _tc_stack_body,
        grid=(B // rb,),
        in_specs=[pl.BlockSpec((rb,), lambda i: (i,)) for _ in range(12)],
        out_specs=pl.BlockSpec((rb, 12), lambda i: (i, 0)),
        out_shape=jax.ShapeDtypeStruct((B, 12), jnp.float32),
    )(*cols)


def _cell(xb4, g, lev, iota):
    rs = (T_RES[lev] - 1, S_RES[lev] - 1, S_RES[lev] - 1, S_RES[lev] - 1)
    rows = g * L + iota
    bli = []
    wts = []
    for c in range(4):
        xv = plsc.load_gather(xb4, [rows, jnp.full((L,), c, jnp.int32)])
        f = xv * jnp.float32(rs[c])
        b = f.astype(jnp.int32)
        b = jnp.minimum(b, rs[c] - 1)
        bli.append(b)
        wts.append(f - b.astype(jnp.float32))
    return bli, wts


def _vertex_indices(bli, lev):
    X = S_RES[lev]
    SY = X
    SZ = X * X
    ST = X * X * X
    base = bli[0] * ST + bli[1] + bli[2] * SY + bli[3] * SZ
    idxs = []
    for i in (0, 1):
        for j in (0, 1):
            for k in (0, 1):
                for l in (0, 1):
                    off = i * ST + j + k * SY + l * SZ
                    idxs.append(base + off)
    return idxs


def _interp(e, wts):
    wt, wx, wy, wz = wts
    a = [e[m] + wt * (e[8 + m] - e[m]) for m in range(8)]
    b = [a[m] + wx * (a[4 + m] - a[m]) for m in range(4)]
    c = [b[m] + wy * (b[2 + m] - b[m]) for m in range(2)]
    return c[0] + wz * (c[1] - c[0])


def _body(x, a0, b0, a1, b1, a2, b2, a3, b3, a4, b4, a5, b5, outs, mask,
          t0a, t0b, t1a, t1b, xb4, idxb, rowa, rowb, outcb, maskb, sem):
    was = (None, None, a2, a3, a4, a5)
    wbs = (None, None, b2, b3, b4, b5)
    wid = lax.axis_index("s") * NC + lax.axis_index("c")
    base = wid * PTS_PER_W
    iota = lax.broadcasted_iota(jnp.int32, (L,), 0)

    pltpu.sync_copy(a0, t0a)
    pltpu.sync_copy(b0, t0b)
    pltpu.sync_copy(a1, t1a)
    pltpu.sync_copy(b1, t1b)
    tabs = ((t0a, t0b), (t1a, t1b))

    def fullv(v):
        return jnp.full((L,), v, dtype=jnp.int32)

    def process_chunk(ci, carry):
        off = base + ci * CHUNK
        pltpu.sync_copy(x.at[pl.ds(off, CHUNK), :], xb4)

        def mask_grp(g, c2):
            rows = g * L + iota
            ok = None
            for c in range(4):
                xv = plsc.load_gather(xb4, [rows, fullv(c)])
                okc = (xv >= jnp.float32(0.0)) & (xv <= jnp.float32(1.0))
                ok = okc if ok is None else (ok & okc)
            maskb[pl.ds(g * L, L)] = jnp.where(ok, jnp.int32(1), jnp.int32(0))
            return c2

        lax.fori_loop(0, GRP, mask_grp, 0)

        for lev in range(NLEV):
            if RESIDENT[lev]:
                ta, tb = tabs[lev]

                def grp_res(g, c2, lev=lev, ta=ta, tb=tb):
                    bli, wts = _cell(xb4, g, lev, iota)
                    vidx = _vertex_indices(bli, lev)
                    sl = pl.ds(g * L, L)
                    for f, tf in ((0, ta), (1, tb)):
                        e = [plsc.load_gather(tf, [vidx[v]])
                             for v in range(16)]
                        o = _interp(e, wts)
                        outcb[2 * lev + f, sl] = o
                    return c2

                lax.fori_loop(0, GRP, grp_res, 0)
            else:
                wa = was[lev]
                wb = wbs[lev]

                def grp_idx(g, c2, lev=lev):
                    bli, _ = _cell(xb4, g, lev, iota)
                    vidx = _vertex_indices(bli, lev)
                    for v in range(16):
                        idxb[pl.ds(v * CHUNK + g * L, L)] = vidx[v]
                    return c2

                lax.fori_loop(0, GRP, grp_idx, 0)
                h1 = pltpu.async_copy(wa.at[idxb], rowa, sem)
                h2 = pltpu.async_copy(wb.at[idxb], rowb, sem)
                h1.wait()
                h2.wait()

                def grp_int(g, c2, lev=lev):
                    _, wts = _cell(xb4, g, lev, iota)
                    sl = pl.ds(g * L, L)
                    for f, rf in ((0, rowa), (1, rowb)):
                        e = [rf[pl.ds(v * CHUNK + g * L, L)]
                             for v in range(16)]
                        o = _interp(e, wts)
                        outcb[2 * lev + f, sl] = o
                    return c2

                lax.fori_loop(0, GRP, grp_int, 0)

        for c in range(12):
            pltpu.sync_copy(outcb.at[c], outs[c].at[pl.ds(off, CHUNK)])
        pltpu.sync_copy(maskb, mask.at[pl.ds(off, CHUNK)])
        return carry

    lax.fori_loop(0, NCHUNK, process_chunk, 0)


_hash_embed = functools.partial(
    pl.kernel,
    out_type=(
        tuple(jax.ShapeDtypeStruct((B,), jnp.float32) for _ in range(12)),
        jax.ShapeDtypeStruct((B,), jnp.int32),
    ),
    mesh=_mesh,
    compiler_params=_cparams,
    scratch_types=[
        pltpu.VMEM((SIZES[0],), jnp.float32),
        pltpu.VMEM((SIZES[0],), jnp.float32),
        pltpu.VMEM((SIZES[1],), jnp.float32),
        pltpu.VMEM((SIZES[1],), jnp.float32),
        pltpu.VMEM((CHUNK, 4), jnp.float32),
        pltpu.VMEM((16 * CHUNK,), jnp.int32),
        pltpu.VMEM((16 * CHUNK,), jnp.float32),
        pltpu.VMEM((16 * CHUNK,), jnp.float32),
        pltpu.VMEM((12, CHUNK), jnp.float32),
        pltpu.VMEM((CHUNK,), jnp.int32),
        pltpu.SemaphoreType.DMA,
    ],
)(_body)


def kernel(x, w0, w1, w2, w3, w4, w5):
    cols = []
    for w in (w0, w1, w2, w3, w4, w5):
        cols.extend(_tc_split(w))
    outs, mask = _hash_embed(x, *cols)
    out = _tc_stack(list(outs))
    return out, mask.astype(bool)
